# baseline (speedup 1.0000x reference)
import jax
import jax.numpy as jnp
from jax.experimental import pallas as pl

_NSPLIT = 4


def _body(x0, x1, x2, x3, emb_ref, o0, o1, o2, o3):
    e = emb_ref[...]
    o0[...] = jnp.dot(x0[...], e, preferred_element_type=jnp.float32)
    o1[...] = jnp.dot(x1[...], e, preferred_element_type=jnp.float32)
    o2[...] = jnp.dot(x2[...], e, preferred_element_type=jnp.float32)
    o3[...] = jnp.dot(x3[...], e, preferred_element_type=jnp.float32)


def kernel(x_seq, emb):
    B, K = x_seq.shape
    H = emb.shape[1]
    BB = 64
    steps = B // (_NSPLIT * BB)
    xspecs = [
        pl.BlockSpec((BB, K), (lambda i, j=j: (j * steps + i, 0)))
        for j in range(_NSPLIT)
    ]
    seg = steps * BB
    ospecs = [pl.BlockSpec((BB, H), lambda i: (i, 0)) for _ in range(_NSPLIT)]
    outs = pl.pallas_call(
        _body,
        grid=(steps,),
        in_specs=xspecs + [pl.BlockSpec((K, H), lambda i: (0, 0))],
        out_specs=ospecs,
        out_shape=[
            jax.ShapeDtypeStruct((seg, H), jnp.float32) for _ in range(_NSPLIT)
        ],
    )(*([x_seq] * _NSPLIT), emb)
    return jnp.concatenate(outs, axis=0)


# speedup vs baseline: 1.0052x; 1.0052x over previous
import jax
import jax.numpy as jnp
from jax.experimental import pallas as pl
from jax.experimental.pallas import tpu as pltpu

_KB = 1024
_NBUF = 8


def _body(x_hbm, emb_ref, out_ref, bufs, tailbuf, sems, tailsem):
    K = x_hbm.shape[1]
    nf = K // _KB
    ts = K - nf * _KB

    def mk(i):
        slot = i % _NBUF
        return pltpu.make_async_copy(
            x_hbm.at[:, pl.ds(i * _KB, _KB)],
            bufs.at[slot],
            sems.at[slot],
        )

    tail_copy = pltpu.make_async_copy(
        x_hbm.at[:, pl.ds(nf * _KB, ts)], tailbuf, tailsem
    )
    tail_copy.start()
    for i in range(min(_NBUF, nf)):
        mk(i).start()

    acc = jnp.zeros(out_ref.shape, jnp.float32)
    for i in range(nf):
        mk(i).wait()
        rhs = emb_ref[i * _KB:(i + 1) * _KB, :]
        acc = acc + jnp.dot(
            bufs[i % _NBUF].astype(jnp.bfloat16),
            rhs.astype(jnp.bfloat16),
            preferred_element_type=jnp.float32,
        )
        if i + _NBUF < nf:
            mk(i + _NBUF).start()

    tail_copy.wait()
    acc = acc + jnp.dot(
        tailbuf[...].astype(jnp.bfloat16),
        emb_ref[nf * _KB:, :].astype(jnp.bfloat16),
        preferred_element_type=jnp.float32,
    )
    out_ref[...] = acc


def kernel(x_seq, emb):
    B, K = x_seq.shape
    H = emb.shape[1]
    nf = K // _KB
    ts = K - nf * _KB
    return pl.pallas_call(
        _body,
        in_specs=[
            pl.BlockSpec(memory_space=pl.ANY),
            pl.BlockSpec(memory_space=pltpu.VMEM),
        ],
        out_specs=pl.BlockSpec(memory_space=pltpu.VMEM),
        out_shape=jax.ShapeDtypeStruct((B, H), jnp.float32),
        scratch_shapes=[
            pltpu.VMEM((_NBUF, B, _KB), jnp.float32),
            pltpu.VMEM((B, ts), jnp.float32),
            pltpu.SemaphoreType.DMA((_NBUF,)),
            pltpu.SemaphoreType.DMA,
        ],
    )(x_seq, emb)
